# baseline (speedup 1.0000x reference)
import functools
import math

import numpy as np
import jax
import jax.numpy as jnp
from jax import lax
from jax.experimental import pallas as pl
from jax.experimental.pallas import tpu as pltpu
from jax.experimental.pallas import tpu_sc as plsc

N = 10000
E = 320000
D = 128
L = 64

NC = 2
NS = 16
NW = NC * NS
CH = 128
SBUF = 2

S = 2
ES = E // S
NCHS = ES // CH
TPWS = NCHS // NW
GRPS = TPWS // SBUF
NTAILS = NCHS - NW * TPWS

ZCH = 80
NZ = N // ZCH

_PERM = np.concatenate([np.arange(w, NCHS, NW) for w in range(NW)]).astype(np.int32)

NBLK = 10
NB = N // NBLK
EB = 3200
GRIDS = ES // EB


def _nodeproj_body(nf_ref, w_ref, out_ref):
    out_ref[...] = jnp.dot(nf_ref[...], w_ref[...],
                           preferred_element_type=jnp.float32)


def _node_proj(nf, w):
    return pl.pallas_call(
        _nodeproj_body,
        grid=(NBLK,),
        in_specs=[
            pl.BlockSpec((NB, D), lambda i: (i, 0)),
            pl.BlockSpec((D, D), lambda i: (0, 0)),
        ],
        out_specs=pl.BlockSpec((NB, D), lambda i: (i, 0)),
        out_shape=jax.ShapeDtypeStruct((N, D), jnp.float32),
    )(nf, w)


def _sc_gather(table, idx3):
    mesh = plsc.VectorSubcoreMesh(core_axis_name="c", subcore_axis_name="s")

    @functools.partial(
        pl.kernel,
        mesh=mesh,
        out_type=jax.ShapeDtypeStruct((ES, D), jnp.float32),
        scratch_types=[
            pltpu.VMEM((TPWS + 1, 1, CH), jnp.int32),
            pltpu.VMEM((SBUF, CH, D), jnp.float32),
            pltpu.VMEM_SHARED((N, D), jnp.float32),
        ] + [pltpu.SemaphoreType.DMA] * (2 * SBUF),
    )
    def k(table_hbm, idx_hbm, out_hbm, idx_v, rows_v, ptab, *sems):
        gsems, osems = sems[:SBUF], sems[SBUF:]
        c = lax.axis_index("c")
        s = lax.axis_index("s")
        wid = s * NC + c
        for t in range((NZ + NS - 1) // NS):
            cid = s + NS * t

            @pl.when(cid < NZ)
            def _():
                pltpu.sync_copy(table_hbm.at[pl.ds(cid * ZCH, ZCH), :],
                                ptab.at[pl.ds(cid * ZCH, ZCH), :])

        offs = wid * TPWS + jnp.minimum(wid, NTAILS)
        pltpu.sync_copy(idx_hbm.at[pl.ds(offs, TPWS + 1)], idx_v)
        plsc.subcore_barrier()

        def grp_body(g, carry):
            handles = []
            for kk in range(SBUF):
                @pl.when(g > 0)
                def _():
                    pltpu.make_async_copy(
                        rows_v.at[kk], out_hbm.at[pl.ds(0, CH), :],
                        osems[kk]).wait()
                t = g * SBUF + kk
                handles.append(pltpu.async_copy(
                    ptab.at[idx_v.at[t, 0]], rows_v.at[kk], gsems[kk]))
            for kk in range(SBUF):
                handles[kk].wait()
                t = g * SBUF + kk
                r = wid + NW * t
                pltpu.async_copy(rows_v.at[kk],
                                 out_hbm.at[pl.ds(r * CH, CH), :], osems[kk])
            return carry

        lax.fori_loop(0, GRPS, grp_body, 0)
        for kk in range(SBUF):
            pltpu.make_async_copy(rows_v.at[kk], out_hbm.at[pl.ds(0, CH), :],
                                  osems[kk]).wait()

        for t in range(GRPS * SBUF, TPWS):
            r = wid + NW * t
            pltpu.async_copy(ptab.at[idx_v.at[t, 0]], rows_v.at[0],
                             gsems[0]).wait()
            pltpu.sync_copy(rows_v.at[0], out_hbm.at[pl.ds(r * CH, CH), :])

        @pl.when(wid < NTAILS)
        def _():
            r = wid + NW * TPWS
            pltpu.async_copy(ptab.at[idx_v.at[TPWS, 0]], rows_v.at[0],
                             gsems[0]).wait()
            pltpu.sync_copy(rows_v.at[0], out_hbm.at[pl.ds(r * CH, CH), :])

    return k(table, idx3)


def _edge_body(g_ref, ef_ref, lat_ref, w9_ref, e9_ref,
               w2_ref, wl_ref, wv9_ref, wp_ref, bp_ref, we_ref, be_ref,
               out_ref):
    zt = w9_ref[...] * e9_ref[...]
    h = (g_ref[...]
         + jnp.dot(ef_ref[...], w2_ref[...], preferred_element_type=jnp.float32)
         + jnp.dot(lat_ref[...], wl_ref[...], preferred_element_type=jnp.float32)
         + jax.lax.dot_general(zt, wv9_ref[...], (((0,), (0,)), ((), ())),
                               preferred_element_type=jnp.float32))
    m = h * jax.nn.sigmoid(h)
    msg = jnp.dot(m, wp_ref[...], preferred_element_type=jnp.float32) + bp_ref[...]
    wts = jnp.dot(lat_ref[...], we_ref[...], preferred_element_type=jnp.float32) + be_ref[...]
    out_ref[...] = msg * wts


def _edge_dense(si, g_e, ef, lat, wig9t, ev9t, w2, wl, wv9, wp, bp, we, be):
    off = si * GRIDS
    return pl.pallas_call(
        _edge_body,
        grid=(GRIDS,),
        in_specs=[
            pl.BlockSpec((EB, D), lambda i: (i, 0)),
            pl.BlockSpec((EB, D), lambda i, o=off: (i + o, 0)),
            pl.BlockSpec((EB, L), lambda i, o=off: (i + o, 0)),
            pl.BlockSpec((9, EB), lambda i, o=off: (0, i + o)),
            pl.BlockSpec((9, EB), lambda i, o=off: (0, i + o)),
            pl.BlockSpec((D, D), lambda i: (0, 0)),
            pl.BlockSpec((L, D), lambda i: (0, 0)),
            pl.BlockSpec((9, D), lambda i: (0, 0)),
            pl.BlockSpec((D, D), lambda i: (0, 0)),
            pl.BlockSpec((1, D), lambda i: (0, 0)),
            pl.BlockSpec((L, D), lambda i: (0, 0)),
            pl.BlockSpec((1, D), lambda i: (0, 0)),
        ],
        out_specs=pl.BlockSpec((EB, D), lambda i: (i, 0)),
        out_shape=jax.ShapeDtypeStruct((ES, D), jnp.float32),
    )(g_e, ef, lat, wig9t, ev9t, w2, wl, wv9, wp, bp, we, be)


def _sc_scatter(weighted, idx3, zeros_rows):
    mesh = plsc.VectorSubcoreMesh(core_axis_name="c", subcore_axis_name="s")

    @functools.partial(
        pl.kernel,
        mesh=mesh,
        out_type=jax.ShapeDtypeStruct((NC * N, D), jnp.float32),
        scratch_types=[
            pltpu.VMEM((TPWS + 1, 1, CH), jnp.int32),
            pltpu.VMEM((SBUF, CH, D), jnp.float32),
            pltpu.VMEM_SHARED((N, D), jnp.float32),
        ] + [pltpu.SemaphoreType.DMA] * (2 * SBUF),
    )
    def k(w_hbm, idx_hbm, z_hbm, out_hbm, idx_v, rows_v, acc, *sems):
        lsems, ssems = sems[:SBUF], sems[SBUF:]
        c = lax.axis_index("c")
        s = lax.axis_index("s")
        wid = s * NC + c
        for t in range((NZ + NS - 1) // NS):
            cid = s + NS * t

            @pl.when(cid < NZ)
            def _():
                pltpu.sync_copy(z_hbm, acc.at[pl.ds(cid * ZCH, ZCH), :])

        offs = wid * TPWS + jnp.minimum(wid, NTAILS)
        pltpu.sync_copy(idx_hbm.at[pl.ds(offs, TPWS + 1)], idx_v)
        plsc.subcore_barrier()

        def grp_body(g, carry):
            handles = []
            for kk in range(SBUF):
                @pl.when(g > 0)
                def _():
                    pltpu.make_async_copy(
                        w_hbm.at[pl.ds(0, CH), :], rows_v.at[kk],
                        ssems[kk]).wait()
                t = g * SBUF + kk
                r = wid + NW * t
                handles.append(pltpu.async_copy(
                    w_hbm.at[pl.ds(r * CH, CH), :], rows_v.at[kk], lsems[kk]))
            for kk in range(SBUF):
                handles[kk].wait()
                t = g * SBUF + kk
                pltpu.async_copy(rows_v.at[kk], acc.at[idx_v.at[t, 0]],
                                 ssems[kk], add=True)
            return carry

        lax.fori_loop(0, GRPS, grp_body, 0)
        for kk in range(SBUF):
            pltpu.make_async_copy(w_hbm.at[pl.ds(0, CH), :], rows_v.at[kk],
                                  ssems[kk]).wait()

        for t in range(GRPS * SBUF, TPWS):
            r = wid + NW * t
            pltpu.sync_copy(w_hbm.at[pl.ds(r * CH, CH), :], rows_v.at[0])
            pltpu.sync_copy(rows_v.at[0], acc.at[idx_v.at[t, 0]], add=True)

        @pl.when(wid < NTAILS)
        def _():
            r = wid + NW * TPWS
            pltpu.sync_copy(w_hbm.at[pl.ds(r * CH, CH), :], rows_v.at[0])
            pltpu.sync_copy(rows_v.at[0], acc.at[idx_v.at[TPWS, 0]], add=True)

        plsc.subcore_barrier()
        for t in range((NZ + NS - 1) // NS):
            cid = s + NS * t

            @pl.when(cid < NZ)
            def _():
                pltpu.sync_copy(acc.at[pl.ds(cid * ZCH, ZCH), :],
                                out_hbm.at[pl.ds(c * N + cid * ZCH, ZCH), :])

    return k(weighted, idx3, zeros_rows)


def _combine_body(nf_ref, oh_ref, woh_ref, *rest, c_old, c_agg):
    p_refs, out_ref = rest[:-1], rest[-1]
    agg = p_refs[0][...]
    for pr in p_refs[1:]:
        agg = agg + pr[...]
    base = c_old * nf_ref[...] + c_agg * agg
    scale = 1.0 + jnp.dot(oh_ref[...], woh_ref[...],
                          preferred_element_type=jnp.float32)
    out_ref[...] = base * scale


def _combine(nf, partials_list, onehot, woh, c_old, c_agg):
    nt = onehot.shape[1]
    p_specs = []
    p_args = []
    for p in partials_list:
        p_specs.append(pl.BlockSpec((NB, D), lambda i: (i, 0)))
        p_specs.append(pl.BlockSpec((NB, D), lambda i: (i + NBLK, 0)))
        p_args.extend([p, p])
    return pl.pallas_call(
        functools.partial(_combine_body, c_old=c_old, c_agg=c_agg),
        grid=(NBLK,),
        in_specs=[
            pl.BlockSpec((NB, D), lambda i: (i, 0)),
            pl.BlockSpec((NB, nt), lambda i: (i, 0)),
            pl.BlockSpec((nt, D), lambda i: (0, 0)),
        ] + p_specs,
        out_specs=pl.BlockSpec((NB, D), lambda i: (i, 0)),
        out_shape=jax.ShapeDtypeStruct((N, D), jnp.float32),
    )(nf, onehot, woh, *p_args)


def kernel(latents, node_features, edge_features, atom_type, node_onehot,
           edge_index, edge_vector, active_edges, wigner_D_all, mole_globals,
           W_tp, W_lat, W_vec, W_glob, W_post, b_post, W_env, b_env, W_oh):
    f32 = jnp.float32
    ec = edge_index[0].astype(jnp.int32)
    idx_segs = []
    for si in range(S):
        seg = lax.slice_in_dim(ec, si * ES, (si + 1) * ES).reshape(NCHS, 1, CH)
        idx_segs.append(jnp.concatenate(
            [seg[_PERM], jnp.zeros((NW - NTAILS, 1, CH), jnp.int32)], axis=0))

    g = jax.nn.sigmoid(mole_globals.astype(f32) @ W_glob.astype(f32))
    w1 = W_tp[:D].astype(f32) * g
    w2 = W_tp[D:].astype(f32) * g
    wl = W_lat.astype(f32) * g
    wv9 = jnp.repeat(W_vec.astype(f32) * g, 3, axis=0)

    wig9t = wigner_D_all.reshape(E, 9).astype(f32).T
    ev9t = jnp.tile(edge_vector.astype(f32).T, (3, 1))

    bf16 = jnp.bfloat16
    ef = edge_features.astype(bf16)
    lat = latents.astype(bf16)
    w2 = w2.astype(bf16)
    wl = wl.astype(bf16)
    wp = W_post.astype(f32)
    bp = b_post.astype(f32).reshape(1, D)
    we = W_env.astype(bf16)
    be = b_env.astype(f32).reshape(1, D)

    p_tab = _node_proj(node_features.astype(f32), w1)
    zeros_rows = jnp.zeros((ZCH, D), dtype=f32)

    partials_list = []
    for si in range(S):
        g_e = _sc_gather(p_tab, idx_segs[si])
        weighted = _edge_dense(si, g_e, ef, lat, wig9t, ev9t,
                               w2, wl, wv9, wp, bp, we, be)
        partials_list.append(_sc_scatter(weighted, idx_segs[si], zeros_rows))

    c_old = 1.0 / math.sqrt(1.25)
    c_new = 0.5 * c_old
    norm = 1.0 / math.sqrt(32.0)
    return _combine(node_features.astype(f32), partials_list,
                    node_onehot.astype(f32), W_oh.astype(f32),
                    c_old, c_new * norm)


# speedup vs baseline: 1.0667x; 1.0667x over previous
import functools
import math

import numpy as np
import jax
import jax.numpy as jnp
from jax import lax
from jax.experimental import pallas as pl
from jax.experimental.pallas import tpu as pltpu
from jax.experimental.pallas import tpu_sc as plsc

N = 10000
E = 320000
D = 128
L = 64

NC = 2
NS = 16
NW = NC * NS
CH = 128
SBUF = 2

S = 2
ES = E // S
NCHS = ES // CH
TPWS = NCHS // NW
GRPS = TPWS // SBUF
NTAILS = NCHS - NW * TPWS

ZCH = 80
NZ = N // ZCH

_PERM = np.concatenate([np.arange(w, NCHS, NW) for w in range(NW)]).astype(np.int32)

NBLK = 10
NB = N // NBLK
EB = 6400
GRIDS = ES // EB


def _nodeproj_body(nf_ref, w_ref, out_ref):
    out_ref[...] = jnp.dot(nf_ref[...], w_ref[...],
                           preferred_element_type=jnp.float32)


def _node_proj(nf, w):
    return pl.pallas_call(
        _nodeproj_body,
        grid=(NBLK,),
        in_specs=[
            pl.BlockSpec((NB, D), lambda i: (i, 0)),
            pl.BlockSpec((D, D), lambda i: (0, 0)),
        ],
        out_specs=pl.BlockSpec((NB, D), lambda i: (i, 0)),
        out_shape=jax.ShapeDtypeStruct((N, D), jnp.float32),
    )(nf, w)


def _sc_gather(table, idx3):
    mesh = plsc.VectorSubcoreMesh(core_axis_name="c", subcore_axis_name="s")

    @functools.partial(
        pl.kernel,
        mesh=mesh,
        out_type=jax.ShapeDtypeStruct((ES, D), jnp.float32),
        scratch_types=[
            pltpu.VMEM((TPWS + 1, 1, CH), jnp.int32),
            pltpu.VMEM((SBUF, CH, D), jnp.float32),
            pltpu.VMEM_SHARED((N, D), jnp.float32),
        ] + [pltpu.SemaphoreType.DMA] * (2 * SBUF),
    )
    def k(table_hbm, idx_hbm, out_hbm, idx_v, rows_v, ptab, *sems):
        gsems, osems = sems[:SBUF], sems[SBUF:]
        c = lax.axis_index("c")
        s = lax.axis_index("s")
        wid = s * NC + c
        for t in range((NZ + NS - 1) // NS):
            cid = s + NS * t

            @pl.when(cid < NZ)
            def _():
                pltpu.sync_copy(table_hbm.at[pl.ds(cid * ZCH, ZCH), :],
                                ptab.at[pl.ds(cid * ZCH, ZCH), :])

        offs = wid * TPWS + jnp.minimum(wid, NTAILS)
        pltpu.sync_copy(idx_hbm.at[pl.ds(offs, TPWS + 1)], idx_v)
        plsc.subcore_barrier()

        def grp_body(g, carry):
            handles = []
            for kk in range(SBUF):
                @pl.when(g > 0)
                def _():
                    pltpu.make_async_copy(
                        rows_v.at[kk], out_hbm.at[pl.ds(0, CH), :],
                        osems[kk]).wait()
                t = g * SBUF + kk
                handles.append(pltpu.async_copy(
                    ptab.at[idx_v.at[t, 0]], rows_v.at[kk], gsems[kk]))
            for kk in range(SBUF):
                handles[kk].wait()
                t = g * SBUF + kk
                r = wid + NW * t
                pltpu.async_copy(rows_v.at[kk],
                                 out_hbm.at[pl.ds(r * CH, CH), :], osems[kk])
            return carry

        lax.fori_loop(0, GRPS, grp_body, 0)
        for kk in range(SBUF):
            pltpu.make_async_copy(rows_v.at[kk], out_hbm.at[pl.ds(0, CH), :],
                                  osems[kk]).wait()

        for t in range(GRPS * SBUF, TPWS):
            r = wid + NW * t
            pltpu.async_copy(ptab.at[idx_v.at[t, 0]], rows_v.at[0],
                             gsems[0]).wait()
            pltpu.sync_copy(rows_v.at[0], out_hbm.at[pl.ds(r * CH, CH), :])

        @pl.when(wid < NTAILS)
        def _():
            r = wid + NW * TPWS
            pltpu.async_copy(ptab.at[idx_v.at[TPWS, 0]], rows_v.at[0],
                             gsems[0]).wait()
            pltpu.sync_copy(rows_v.at[0], out_hbm.at[pl.ds(r * CH, CH), :])

    return k(table, idx3)


def _edge_body(g_ref, ef_ref, lat_ref, w9_ref, e9_ref,
               w2_ref, wl_ref, wv9_ref, wp_ref, bp_ref, we_ref, be_ref,
               out_ref):
    zt = w9_ref[...] * e9_ref[...]
    h = (g_ref[...]
         + jnp.dot(ef_ref[...], w2_ref[...], preferred_element_type=jnp.float32)
         + jnp.dot(lat_ref[...], wl_ref[...], preferred_element_type=jnp.float32)
         + jax.lax.dot_general(zt, wv9_ref[...], (((0,), (0,)), ((), ())),
                               preferred_element_type=jnp.float32))
    m = h * jax.nn.sigmoid(h)
    msg = jnp.dot(m, wp_ref[...], preferred_element_type=jnp.float32) + bp_ref[...]
    wts = jnp.dot(lat_ref[...], we_ref[...], preferred_element_type=jnp.float32) + be_ref[...]
    out_ref[...] = msg * wts


def _edge_dense(si, g_e, ef, lat, wig9t, ev9t, w2, wl, wv9, wp, bp, we, be):
    off = si * GRIDS
    return pl.pallas_call(
        _edge_body,
        grid=(GRIDS,),
        in_specs=[
            pl.BlockSpec((EB, D), lambda i: (i, 0)),
            pl.BlockSpec((EB, D), lambda i, o=off: (i + o, 0)),
            pl.BlockSpec((EB, L), lambda i, o=off: (i + o, 0)),
            pl.BlockSpec((9, EB), lambda i, o=off: (0, i + o)),
            pl.BlockSpec((9, EB), lambda i, o=off: (0, i + o)),
            pl.BlockSpec((D, D), lambda i: (0, 0)),
            pl.BlockSpec((L, D), lambda i: (0, 0)),
            pl.BlockSpec((9, D), lambda i: (0, 0)),
            pl.BlockSpec((D, D), lambda i: (0, 0)),
            pl.BlockSpec((1, D), lambda i: (0, 0)),
            pl.BlockSpec((L, D), lambda i: (0, 0)),
            pl.BlockSpec((1, D), lambda i: (0, 0)),
        ],
        out_specs=pl.BlockSpec((EB, D), lambda i: (i, 0)),
        out_shape=jax.ShapeDtypeStruct((ES, D), jnp.float32),
    )(g_e, ef, lat, wig9t, ev9t, w2, wl, wv9, wp, bp, we, be)


def _sc_scatter(weighted, idx3, zeros_rows):
    mesh = plsc.VectorSubcoreMesh(core_axis_name="c", subcore_axis_name="s")

    @functools.partial(
        pl.kernel,
        mesh=mesh,
        out_type=jax.ShapeDtypeStruct((NC * N, D), jnp.float32),
        scratch_types=[
            pltpu.VMEM((TPWS + 1, 1, CH), jnp.int32),
            pltpu.VMEM((SBUF, CH, D), jnp.float32),
            pltpu.VMEM_SHARED((N, D), jnp.float32),
        ] + [pltpu.SemaphoreType.DMA] * (2 * SBUF),
    )
    def k(w_hbm, idx_hbm, z_hbm, out_hbm, idx_v, rows_v, acc, *sems):
        lsems, ssems = sems[:SBUF], sems[SBUF:]
        c = lax.axis_index("c")
        s = lax.axis_index("s")
        wid = s * NC + c
        for t in range((NZ + NS - 1) // NS):
            cid = s + NS * t

            @pl.when(cid < NZ)
            def _():
                pltpu.sync_copy(z_hbm, acc.at[pl.ds(cid * ZCH, ZCH), :])

        offs = wid * TPWS + jnp.minimum(wid, NTAILS)
        pltpu.sync_copy(idx_hbm.at[pl.ds(offs, TPWS + 1)], idx_v)
        plsc.subcore_barrier()

        def grp_body(g, carry):
            handles = []
            for kk in range(SBUF):
                @pl.when(g > 0)
                def _():
                    pltpu.make_async_copy(
                        w_hbm.at[pl.ds(0, CH), :], rows_v.at[kk],
                        ssems[kk]).wait()
                t = g * SBUF + kk
                r = wid + NW * t
                handles.append(pltpu.async_copy(
                    w_hbm.at[pl.ds(r * CH, CH), :], rows_v.at[kk], lsems[kk]))
            for kk in range(SBUF):
                handles[kk].wait()
                t = g * SBUF + kk
                pltpu.async_copy(rows_v.at[kk], acc.at[idx_v.at[t, 0]],
                                 ssems[kk], add=True)
            return carry

        lax.fori_loop(0, GRPS, grp_body, 0)
        for kk in range(SBUF):
            pltpu.make_async_copy(w_hbm.at[pl.ds(0, CH), :], rows_v.at[kk],
                                  ssems[kk]).wait()

        for t in range(GRPS * SBUF, TPWS):
            r = wid + NW * t
            pltpu.sync_copy(w_hbm.at[pl.ds(r * CH, CH), :], rows_v.at[0])
            pltpu.sync_copy(rows_v.at[0], acc.at[idx_v.at[t, 0]], add=True)

        @pl.when(wid < NTAILS)
        def _():
            r = wid + NW * TPWS
            pltpu.sync_copy(w_hbm.at[pl.ds(r * CH, CH), :], rows_v.at[0])
            pltpu.sync_copy(rows_v.at[0], acc.at[idx_v.at[TPWS, 0]], add=True)

        plsc.subcore_barrier()
        for t in range((NZ + NS - 1) // NS):
            cid = s + NS * t

            @pl.when(cid < NZ)
            def _():
                pltpu.sync_copy(acc.at[pl.ds(cid * ZCH, ZCH), :],
                                out_hbm.at[pl.ds(c * N + cid * ZCH, ZCH), :])

    return k(weighted, idx3, zeros_rows)


def _combine_body(nf_ref, oh_ref, woh_ref, *rest, c_old, c_agg):
    p_refs, out_ref = rest[:-1], rest[-1]
    agg = p_refs[0][...]
    for pr in p_refs[1:]:
        agg = agg + pr[...]
    base = c_old * nf_ref[...] + c_agg * agg
    scale = 1.0 + jnp.dot(oh_ref[...], woh_ref[...],
                          preferred_element_type=jnp.float32)
    out_ref[...] = base * scale


def _combine(nf, partials_list, onehot, woh, c_old, c_agg):
    nt = onehot.shape[1]
    p_specs = []
    p_args = []
    for p in partials_list:
        p_specs.append(pl.BlockSpec((NB, D), lambda i: (i, 0)))
        p_specs.append(pl.BlockSpec((NB, D), lambda i: (i + NBLK, 0)))
        p_args.extend([p, p])
    return pl.pallas_call(
        functools.partial(_combine_body, c_old=c_old, c_agg=c_agg),
        grid=(NBLK,),
        in_specs=[
            pl.BlockSpec((NB, D), lambda i: (i, 0)),
            pl.BlockSpec((NB, nt), lambda i: (i, 0)),
            pl.BlockSpec((nt, D), lambda i: (0, 0)),
        ] + p_specs,
        out_specs=pl.BlockSpec((NB, D), lambda i: (i, 0)),
        out_shape=jax.ShapeDtypeStruct((N, D), jnp.float32),
    )(nf, onehot, woh, *p_args)


def kernel(latents, node_features, edge_features, atom_type, node_onehot,
           edge_index, edge_vector, active_edges, wigner_D_all, mole_globals,
           W_tp, W_lat, W_vec, W_glob, W_post, b_post, W_env, b_env, W_oh):
    f32 = jnp.float32
    ec = edge_index[0].astype(jnp.int32)
    idx_segs = []
    for si in range(S):
        seg = lax.slice_in_dim(ec, si * ES, (si + 1) * ES).reshape(NCHS, 1, CH)
        idx_segs.append(jnp.concatenate(
            [seg[_PERM], jnp.zeros((NW - NTAILS, 1, CH), jnp.int32)], axis=0))

    g = jax.nn.sigmoid(mole_globals.astype(f32) @ W_glob.astype(f32))
    w1 = W_tp[:D].astype(f32) * g
    w2 = W_tp[D:].astype(f32) * g
    wl = W_lat.astype(f32) * g
    wv9 = jnp.repeat(W_vec.astype(f32) * g, 3, axis=0)

    wig9t = wigner_D_all.reshape(E, 9).astype(f32).T
    ev9t = jnp.tile(edge_vector.astype(f32).T, (3, 1))

    ef = edge_features.astype(f32)
    lat = latents.astype(f32)
    wp = W_post.astype(f32)
    bp = b_post.astype(f32).reshape(1, D)
    we = W_env.astype(f32)
    be = b_env.astype(f32).reshape(1, D)

    p_tab = _node_proj(node_features.astype(f32), w1)
    zeros_rows = jnp.zeros((ZCH, D), dtype=f32)

    partials_list = []
    for si in range(S):
        g_e = _sc_gather(p_tab, idx_segs[si])
        weighted = _edge_dense(si, g_e, ef, lat, wig9t, ev9t,
                               w2, wl, wv9, wp, bp, we, be)
        partials_list.append(_sc_scatter(weighted, idx_segs[si], zeros_rows))

    c_old = 1.0 / math.sqrt(1.25)
    c_new = 0.5 * c_old
    norm = 1.0 / math.sqrt(32.0)
    return _combine(node_features.astype(f32), partials_list,
                    node_onehot.astype(f32), W_oh.astype(f32),
                    c_old, c_new * norm)


# speedup vs baseline: 1.1223x; 1.0521x over previous
import functools
import math

import numpy as np
import jax
import jax.numpy as jnp
from jax import lax
from jax.experimental import pallas as pl
from jax.experimental.pallas import tpu as pltpu
from jax.experimental.pallas import tpu_sc as plsc

N = 10000
E = 320000
D = 128
L = 64

NC = 2
NS = 16
NW = NC * NS
CH = 128
SBUF = 2

SEG_EDGES = (128000, 192000)
S = len(SEG_EDGES)

ZCH = 80
NZ = N // ZCH

NBLK = 10
NB = N // NBLK
EB = 6400


class _Seg:
    def __init__(self, start, n_edges):
        self.start = start
        self.es = n_edges
        self.nchs = n_edges // CH
        self.tpws = self.nchs // NW
        self.grps = self.tpws // SBUF
        self.ntails = self.nchs - NW * self.tpws
        self.grids = n_edges // EB
        self.blk_off = start // EB
        self.perm = np.concatenate(
            [np.arange(w, self.nchs, NW) for w in range(NW)]).astype(np.int32)


_SEGS = []
_off = 0
for _n in SEG_EDGES:
    _SEGS.append(_Seg(_off, _n))
    _off += _n
assert _off == E and all(sg.es % EB == 0 and sg.es % CH == 0 for sg in _SEGS)


def _nodeproj_body(nf_ref, w_ref, out_ref):
    out_ref[...] = jnp.dot(nf_ref[...], w_ref[...],
                           preferred_element_type=jnp.float32)


def _node_proj(nf, w):
    return pl.pallas_call(
        _nodeproj_body,
        grid=(NBLK,),
        in_specs=[
            pl.BlockSpec((NB, D), lambda i: (i, 0)),
            pl.BlockSpec((D, D), lambda i: (0, 0)),
        ],
        out_specs=pl.BlockSpec((NB, D), lambda i: (i, 0)),
        out_shape=jax.ShapeDtypeStruct((N, D), jnp.float32),
    )(nf, w)


def _sc_gather(sg, table, idx3):
    mesh = plsc.VectorSubcoreMesh(core_axis_name="c", subcore_axis_name="s")

    @functools.partial(
        pl.kernel,
        mesh=mesh,
        out_type=jax.ShapeDtypeStruct((sg.es, D), jnp.float32),
        scratch_types=[
            pltpu.VMEM((sg.tpws + 1, 1, CH), jnp.int32),
            pltpu.VMEM((SBUF, CH, D), jnp.float32),
            pltpu.VMEM_SHARED((N, D), jnp.float32),
        ] + [pltpu.SemaphoreType.DMA] * (2 * SBUF),
    )
    def k(table_hbm, idx_hbm, out_hbm, idx_v, rows_v, ptab, *sems):
        gsems, osems = sems[:SBUF], sems[SBUF:]
        c = lax.axis_index("c")
        s = lax.axis_index("s")
        wid = s * NC + c
        for t in range((NZ + NS - 1) // NS):
            cid = s + NS * t

            @pl.when(cid < NZ)
            def _():
                pltpu.sync_copy(table_hbm.at[pl.ds(cid * ZCH, ZCH), :],
                                ptab.at[pl.ds(cid * ZCH, ZCH), :])

        offs = wid * sg.tpws + jnp.minimum(wid, sg.ntails)
        pltpu.sync_copy(idx_hbm.at[pl.ds(offs, sg.tpws + 1)], idx_v)
        plsc.subcore_barrier()

        def grp_body(g, carry):
            handles = []
            for kk in range(SBUF):
                @pl.when(g > 0)
                def _():
                    pltpu.make_async_copy(
                        rows_v.at[kk], out_hbm.at[pl.ds(0, CH), :],
                        osems[kk]).wait()
                t = g * SBUF + kk
                handles.append(pltpu.async_copy(
                    ptab.at[idx_v.at[t, 0]], rows_v.at[kk], gsems[kk]))
            for kk in range(SBUF):
                handles[kk].wait()
                t = g * SBUF + kk
                r = wid + NW * t
                pltpu.async_copy(rows_v.at[kk],
                                 out_hbm.at[pl.ds(r * CH, CH), :], osems[kk])
            return carry

        lax.fori_loop(0, sg.grps, grp_body, 0)
        for kk in range(SBUF):
            pltpu.make_async_copy(rows_v.at[kk], out_hbm.at[pl.ds(0, CH), :],
                                  osems[kk]).wait()

        for t in range(sg.grps * SBUF, sg.tpws):
            r = wid + NW * t
            pltpu.async_copy(ptab.at[idx_v.at[t, 0]], rows_v.at[0],
                             gsems[0]).wait()
            pltpu.sync_copy(rows_v.at[0], out_hbm.at[pl.ds(r * CH, CH), :])

        @pl.when(wid < sg.ntails)
        def _():
            r = wid + NW * sg.tpws
            pltpu.async_copy(ptab.at[idx_v.at[sg.tpws, 0]], rows_v.at[0],
                             gsems[0]).wait()
            pltpu.sync_copy(rows_v.at[0], out_hbm.at[pl.ds(r * CH, CH), :])

    return k(table, idx3)


def _edge_body(g_ref, ef_ref, lat_ref, w9_ref, e9_ref,
               w2_ref, wl_ref, wv9_ref, wp_ref, bp_ref, we_ref, be_ref,
               out_ref):
    zt = w9_ref[...] * e9_ref[...]
    h = (g_ref[...]
         + jnp.dot(ef_ref[...], w2_ref[...], preferred_element_type=jnp.float32)
         + jnp.dot(lat_ref[...], wl_ref[...], preferred_element_type=jnp.float32)
         + jax.lax.dot_general(zt, wv9_ref[...], (((0,), (0,)), ((), ())),
                               preferred_element_type=jnp.float32))
    m = h * jax.nn.sigmoid(h)
    msg = jnp.dot(m, wp_ref[...], preferred_element_type=jnp.float32) + bp_ref[...]
    wts = jnp.dot(lat_ref[...], we_ref[...], preferred_element_type=jnp.float32) + be_ref[...]
    out_ref[...] = msg * wts


def _edge_dense(sg, g_e, ef, lat, wig9t, ev9t, w2, wl, wv9, wp, bp, we, be):
    off = sg.blk_off
    return pl.pallas_call(
        _edge_body,
        grid=(sg.grids,),
        in_specs=[
            pl.BlockSpec((EB, D), lambda i: (i, 0)),
            pl.BlockSpec((EB, D), lambda i, o=off: (i + o, 0)),
            pl.BlockSpec((EB, L), lambda i, o=off: (i + o, 0)),
            pl.BlockSpec((9, EB), lambda i, o=off: (0, i + o)),
            pl.BlockSpec((9, EB), lambda i, o=off: (0, i + o)),
            pl.BlockSpec((D, D), lambda i: (0, 0)),
            pl.BlockSpec((L, D), lambda i: (0, 0)),
            pl.BlockSpec((9, D), lambda i: (0, 0)),
            pl.BlockSpec((D, D), lambda i: (0, 0)),
            pl.BlockSpec((1, D), lambda i: (0, 0)),
            pl.BlockSpec((L, D), lambda i: (0, 0)),
            pl.BlockSpec((1, D), lambda i: (0, 0)),
        ],
        out_specs=pl.BlockSpec((EB, D), lambda i: (i, 0)),
        out_shape=jax.ShapeDtypeStruct((sg.es, D), jnp.float32),
    )(g_e, ef, lat, wig9t, ev9t, w2, wl, wv9, wp, bp, we, be)


def _sc_scatter(sg, weighted, idx3, zeros_rows):
    mesh = plsc.VectorSubcoreMesh(core_axis_name="c", subcore_axis_name="s")

    @functools.partial(
        pl.kernel,
        mesh=mesh,
        out_type=jax.ShapeDtypeStruct((NC * N, D), jnp.float32),
        scratch_types=[
            pltpu.VMEM((sg.tpws + 1, 1, CH), jnp.int32),
            pltpu.VMEM((SBUF, CH, D), jnp.float32),
            pltpu.VMEM_SHARED((N, D), jnp.float32),
        ] + [pltpu.SemaphoreType.DMA] * (2 * SBUF),
    )
    def k(w_hbm, idx_hbm, z_hbm, out_hbm, idx_v, rows_v, acc, *sems):
        lsems, ssems = sems[:SBUF], sems[SBUF:]
        c = lax.axis_index("c")
        s = lax.axis_index("s")
        wid = s * NC + c
        for t in range((NZ + NS - 1) // NS):
            cid = s + NS * t

            @pl.when(cid < NZ)
            def _():
                pltpu.sync_copy(z_hbm, acc.at[pl.ds(cid * ZCH, ZCH), :])

        offs = wid * sg.tpws + jnp.minimum(wid, sg.ntails)
        pltpu.sync_copy(idx_hbm.at[pl.ds(offs, sg.tpws + 1)], idx_v)
        plsc.subcore_barrier()

        def grp_body(g, carry):
            handles = []
            for kk in range(SBUF):
                @pl.when(g > 0)
                def _():
                    pltpu.make_async_copy(
                        w_hbm.at[pl.ds(0, CH), :], rows_v.at[kk],
                        ssems[kk]).wait()
                t = g * SBUF + kk
                r = wid + NW * t
                handles.append(pltpu.async_copy(
                    w_hbm.at[pl.ds(r * CH, CH), :], rows_v.at[kk], lsems[kk]))
            for kk in range(SBUF):
                handles[kk].wait()
                t = g * SBUF + kk
                pltpu.async_copy(rows_v.at[kk], acc.at[idx_v.at[t, 0]],
                                 ssems[kk], add=True)
            return carry

        lax.fori_loop(0, sg.grps, grp_body, 0)
        for kk in range(SBUF):
            pltpu.make_async_copy(w_hbm.at[pl.ds(0, CH), :], rows_v.at[kk],
                                  ssems[kk]).wait()

        for t in range(sg.grps * SBUF, sg.tpws):
            r = wid + NW * t
            pltpu.sync_copy(w_hbm.at[pl.ds(r * CH, CH), :], rows_v.at[0])
            pltpu.sync_copy(rows_v.at[0], acc.at[idx_v.at[t, 0]], add=True)

        @pl.when(wid < sg.ntails)
        def _():
            r = wid + NW * sg.tpws
            pltpu.sync_copy(w_hbm.at[pl.ds(r * CH, CH), :], rows_v.at[0])
            pltpu.sync_copy(rows_v.at[0], acc.at[idx_v.at[sg.tpws, 0]], add=True)

        plsc.subcore_barrier()
        for t in range((NZ + NS - 1) // NS):
            cid = s + NS * t

            @pl.when(cid < NZ)
            def _():
                pltpu.sync_copy(acc.at[pl.ds(cid * ZCH, ZCH), :],
                                out_hbm.at[pl.ds(c * N + cid * ZCH, ZCH), :])

    return k(weighted, idx3, zeros_rows)


def _combine_body(nf_ref, oh_ref, woh_ref, *rest, c_old, c_agg):
    p_refs, out_ref = rest[:-1], rest[-1]
    agg = p_refs[0][...]
    for pr in p_refs[1:]:
        agg = agg + pr[...]
    base = c_old * nf_ref[...] + c_agg * agg
    scale = 1.0 + jnp.dot(oh_ref[...], woh_ref[...],
                          preferred_element_type=jnp.float32)
    out_ref[...] = base * scale


def _combine(nf, partials_list, onehot, woh, c_old, c_agg):
    nt = onehot.shape[1]
    p_specs = []
    p_args = []
    for p in partials_list:
        p_specs.append(pl.BlockSpec((NB, D), lambda i: (i, 0)))
        p_specs.append(pl.BlockSpec((NB, D), lambda i: (i + NBLK, 0)))
        p_args.extend([p, p])
    return pl.pallas_call(
        functools.partial(_combine_body, c_old=c_old, c_agg=c_agg),
        grid=(NBLK,),
        in_specs=[
            pl.BlockSpec((NB, D), lambda i: (i, 0)),
            pl.BlockSpec((NB, nt), lambda i: (i, 0)),
            pl.BlockSpec((nt, D), lambda i: (0, 0)),
        ] + p_specs,
        out_specs=pl.BlockSpec((NB, D), lambda i: (i, 0)),
        out_shape=jax.ShapeDtypeStruct((N, D), jnp.float32),
    )(nf, onehot, woh, *p_args)


def kernel(latents, node_features, edge_features, atom_type, node_onehot,
           edge_index, edge_vector, active_edges, wigner_D_all, mole_globals,
           W_tp, W_lat, W_vec, W_glob, W_post, b_post, W_env, b_env, W_oh):
    f32 = jnp.float32
    ec = edge_index[0].astype(jnp.int32)
    idx_segs = []
    for sg in _SEGS:
        seg = lax.slice_in_dim(ec, sg.start, sg.start + sg.es)
        seg = seg.reshape(sg.nchs, 1, CH)
        idx_segs.append(jnp.concatenate(
            [seg[sg.perm], jnp.zeros((NW - sg.ntails, 1, CH), jnp.int32)],
            axis=0))

    g = jax.nn.sigmoid(mole_globals.astype(f32) @ W_glob.astype(f32))
    w1 = W_tp[:D].astype(f32) * g
    w2 = W_tp[D:].astype(f32) * g
    wl = W_lat.astype(f32) * g
    wv9 = jnp.repeat(W_vec.astype(f32) * g, 3, axis=0)

    wig9t = wigner_D_all.reshape(E, 9).astype(f32).T
    ev9t = jnp.tile(edge_vector.astype(f32).T, (3, 1))

    ef = edge_features.astype(f32)
    lat = latents.astype(f32)
    wp = W_post.astype(f32)
    bp = b_post.astype(f32).reshape(1, D)
    we = W_env.astype(f32)
    be = b_env.astype(f32).reshape(1, D)

    p_tab = _node_proj(node_features.astype(f32), w1)
    zeros_rows = jnp.zeros((ZCH, D), dtype=f32)

    partials_list = []
    for si, sg in enumerate(_SEGS):
        g_e = _sc_gather(sg, p_tab, idx_segs[si])
        weighted = _edge_dense(sg, g_e, ef, lat, wig9t, ev9t,
                               w2, wl, wv9, wp, bp, we, be)
        partials_list.append(_sc_scatter(sg, weighted, idx_segs[si],
                                         zeros_rows))

    c_old = 1.0 / math.sqrt(1.25)
    c_new = 0.5 * c_old
    norm = 1.0 / math.sqrt(32.0)
    return _combine(node_features.astype(f32), partials_list,
                    node_onehot.astype(f32), W_oh.astype(f32),
                    c_old, c_new * norm)
